# baseline (speedup 1.0000x reference)
import functools

import jax
import jax.numpy as jnp
from jax import lax
from jax.experimental import pallas as pl
from jax.experimental.pallas import tpu as pltpu
from jax.experimental.pallas import tpu_sc as plsc

_NC = 2
_NS = 16
_NW = _NC * _NS
_LANES = 16


def _sc_scores(pos_u, pos_v, neg_t, u_rm, v_rm, B, K, D, chunk):
    per_w = B // _NW
    nchunks = per_w // chunk
    ngroups = chunk // _LANES
    mesh = plsc.VectorSubcoreMesh(core_axis_name="c", subcore_axis_name="s")

    @functools.partial(
        pl.kernel,
        out_type=jax.ShapeDtypeStruct(((1 + K) * B,), jnp.float32),
        mesh=mesh,
        compiler_params=pltpu.CompilerParams(needs_layout_passes=False),
        scratch_types=[
            pltpu.VMEM((per_w,), jnp.int32),
            pltpu.VMEM((per_w,), jnp.int32),
            pltpu.VMEM((K, per_w), jnp.int32),
            pltpu.VMEM((chunk, 8, D), jnp.float32),
            pltpu.VMEM((chunk, 8, D), jnp.float32),
            pltpu.VMEM((K * chunk, 8, D), jnp.float32),
            pltpu.VMEM((1 + K, per_w), jnp.float32),
            pltpu.SemaphoreType.DMA,
        ],
    )
    def scores_kernel(pos_u_hbm, pos_v_hbm, neg_t_hbm, u_hbm, v_hbm, out_hbm,
                      aidx_u, aidx_v, aidx_n, u_blk, v_blk, n_blk, scores,
                      sem):
        wid = lax.axis_index("s") * _NC + lax.axis_index("c")
        base = wid * per_w
        lane = lax.iota(jnp.int32, _LANES)
        zero16 = jnp.zeros((_LANES,), jnp.float32)

        pltpu.sync_copy(pos_u_hbm.at[pl.ds(pl.multiple_of(base, per_w),
                                           per_w)], aidx_u)
        pltpu.sync_copy(pos_v_hbm.at[pl.ds(pl.multiple_of(base, per_w),
                                           per_w)], aidx_v)
        pltpu.sync_copy(neg_t_hbm.at[:, pl.ds(pl.multiple_of(base, per_w),
                                              per_w)], aidx_n)

        def row_group(tbl, i):
            i8 = pl.multiple_of((i >> 3) * 8, 8)
            return tbl.at[pl.ds(i8, 8), :]


        def fire_n(c):
            @pl.when(c < nchunks)
            def _():
                nbo = 0
                coff = c * chunk
                for g in range(ngroups):
                    sl = pl.ds(coff + g * _LANES, _LANES)
                    in_vecs = [aidx_n[k, sl] for k in range(K)]
                    for i in range(_LANES):
                        e = g * _LANES + i
                        for k in range(K):
                            pltpu.async_copy(row_group(v_hbm, in_vecs[k][i]),
                                             n_blk.at[nbo + k * chunk + e],
                                             sem)

        def fire_uv(c):
            @pl.when(c < nchunks)
            def _():
                coff = c * chunk
                for g in range(ngroups):
                    sl = pl.ds(coff + g * _LANES, _LANES)
                    iu_vec = aidx_u[sl]
                    iv_vec = aidx_v[sl]
                    for i in range(_LANES):
                        e = g * _LANES + i
                        pltpu.async_copy(row_group(u_hbm, iu_vec[i]),
                                         u_blk.at[e], sem)
                        pltpu.async_copy(row_group(v_hbm, iv_vec[i]),
                                         v_blk.at[e], sem)

        def chunk_body(c, _):
            bo = 0
            nbo = 0
            coff = c * chunk
            for e in range(chunk):
                pltpu.make_async_copy(u_hbm.at[pl.ds(0, 8), :],
                                      u_blk.at[e], sem).wait()
                pltpu.make_async_copy(u_hbm.at[pl.ds(0, 8), :],
                                      v_blk.at[e], sem).wait()
                for k in range(K):
                    pltpu.make_async_copy(
                        u_hbm.at[pl.ds(0, 8), :],
                        n_blk.at[nbo + k * chunk + e], sem).wait()

            for g in range(ngroups):
                sl = pl.ds(coff + g * _LANES, _LANES)
                evec = g * _LANES + lane
                ru = aidx_u[sl] & 7
                rv = aidx_v[sl] & 7
                rn = [aidx_n[k, sl] & 7 for k in range(K)]
                accs = [zero16] * (1 + K)
                for d in range(D):
                    dv = jnp.full((_LANES,), d, jnp.int32)
                    u_col = plsc.load_gather(u_blk, [evec, ru, dv])
                    accs[0] = accs[0] + u_col * plsc.load_gather(
                        v_blk, [evec, rv, dv])
                    for k in range(K):
                        accs[1 + k] = accs[1 + k] + u_col * plsc.load_gather(
                            n_blk, [nbo + k * chunk + evec, rn[k], dv])
                for r in range(1 + K):
                    scores[r, sl] = accs[r]
            fire_n(c + 1)
            fire_uv(c + 1)
            return 0

        fire_n(0)
        fire_uv(0)
        lax.fori_loop(0, nchunks, chunk_body, 0)
        for r in range(1 + K):
            pltpu.sync_copy(scores.at[r],
                            out_hbm.at[pl.ds(r * B + base, per_w)])

    return scores_kernel(pos_u, pos_v, neg_t, u_rm, v_rm)


def _loss_body(s_ref, o_ref):
    s = s_ref[...]
    pos = s[0:1, :]
    neg = s[1:, :]

    def logsig(x):
        return jnp.minimum(x, 0.0) - jnp.log1p(jnp.exp(-jnp.abs(x)))

    total = jnp.sum(logsig(pos)) + jnp.sum(logsig(-neg))
    o_ref[...] = (-total).reshape(1, 1)


def kernel(pos_u, pos_v, neg_v, u_table, v_table):
    B = pos_u.shape[0]
    K = neg_v.shape[1]
    V, D = u_table.shape
    pos_u = pos_u.astype(jnp.int32)
    pos_v = pos_v.astype(jnp.int32)
    neg_t = neg_v.astype(jnp.int32).T

    scores = _sc_scores(pos_u, pos_v, neg_t, u_table, v_table, B, K, D,
                        chunk=16).reshape(1 + K, B)

    loss = pl.pallas_call(
        _loss_body,
        out_shape=jax.ShapeDtypeStruct((1, 1), jnp.float32),
    )(scores)
    return loss[0, 0]
